# baseline (speedup 1.0000x reference)
import functools

import jax
import jax.numpy as jnp
from jax import lax
from jax.experimental import pallas as pl
from jax.experimental.pallas import tpu as pltpu
from jax.experimental.pallas import tpu_sc as plsc

NUM_TOKENS = 1024
LATENT = 64
SPOS = 1024
CHUNK = 256
N_ROWS = 16 * SPOS
SC_WORKERS = 32
ROWS_PER_W = N_ROWS // SC_WORKERS


def _vq_idx_block(xb_ref, cb_ref, csq_ref, idx_ref):
    cb = cb_ref[...]
    csq = csq_ref[...]
    for j in range(SPOS // CHUNK):
        cols = pl.ds(j * CHUNK, CHUNK)
        xb = xb_ref[0, :, cols]
        prod = lax.dot_general(xb, cb, (((0,), (1,)), ((), ())),
                               preferred_element_type=jnp.float32)
        xsq = jnp.sum(xb * xb, axis=0)[:, None]
        dist = (xsq - 2.0 * prod) + csq
        idx_ref[0, 0, cols] = jnp.argmin(dist, axis=1).astype(jnp.int32)


def _sc_gather(cb_hbm, idx_hbm, out_hbm, idx_v, rows_v, sem):
    wid = lax.axis_index("s") * 2 + lax.axis_index("c")
    base = wid * ROWS_PER_W
    pltpu.sync_copy(idx_hbm.at[pl.ds(base, ROWS_PER_W)], idx_v)
    pltpu.async_copy(cb_hbm.at[idx_v], rows_v, sem).wait()
    pltpu.sync_copy(rows_v, out_hbm.at[pl.ds(base, ROWS_PER_W)])


@functools.partial(jax.jit, static_argnames=("interpret",))
def kernel(x, codebook, interpret=False):
    b, c, h, w = x.shape
    hw = h * w
    xr = x.reshape(b, c, hw)
    csq = jnp.sum(codebook * codebook, axis=-1)[None, :]
    idx_r = pl.pallas_call(
        _vq_idx_block,
        grid=(b,),
        in_specs=[
            pl.BlockSpec((1, LATENT, hw), lambda i: (i, 0, 0)),
            pl.BlockSpec((NUM_TOKENS, LATENT), lambda i: (0, 0)),
            pl.BlockSpec((1, NUM_TOKENS), lambda i: (0, 0)),
        ],
        out_specs=pl.BlockSpec((1, 1, hw), lambda i: (i, 0, 0)),
        out_shape=jax.ShapeDtypeStruct((b, 1, hw), jnp.int32),
        interpret=interpret,
    )(xr, codebook, csq)
    idx_flat = idx_r.reshape(N_ROWS)
    cb_pad = jnp.pad(codebook, ((0, 0), (0, 128 - LATENT)))
    sc_fn = pl.kernel(
        _sc_gather,
        mesh=plsc.VectorSubcoreMesh(core_axis_name="c", subcore_axis_name="s"),
        out_type=jax.ShapeDtypeStruct((N_ROWS, 128), jnp.float32),
        scratch_types=[
            pltpu.VMEM((ROWS_PER_W,), jnp.int32),
            pltpu.VMEM((ROWS_PER_W, 128), jnp.float32),
            pltpu.SemaphoreType.DMA,
        ],
    )
    codes_flat = sc_fn(cb_pad, idx_flat)[:, :LATENT]
    codes = jnp.transpose(codes_flat.reshape(b, hw, c), (0, 2, 1)).reshape(b, c, h, w)
    indices = idx_r.reshape(b, h, w)
    return (codes, indices)


# speedup vs baseline: 1.5310x; 1.5310x over previous
import jax
import jax.numpy as jnp
from jax import lax
from jax.experimental import pallas as pl

NUM_TOKENS = 1024
LATENT = 64
SPOS = 1024
CHUNK = 256
BPS = 4


def _vq_block(xb_ref, cbm2_ref, csq_ref, cbhit_ref, cblot_ref, codes_ref, idx_ref):
    cbm2 = cbm2_ref[...]
    csq = csq_ref[...]
    cbhit = cbhit_ref[...]
    cblot = cblot_ref[...]
    for bb in range(BPS):
        for j in range(SPOS // CHUNK):
            cols = pl.ds(j * CHUNK, CHUNK)
            xb = xb_ref[bb, :, cols]
            prodm2 = lax.dot_general(xb, cbm2, (((0,), (1,)), ((), ())),
                                     preferred_element_type=jnp.float32)
            xsq = jnp.sum(xb * xb, axis=0)[:, None]
            dist = (xsq + prodm2) + csq
            idx = jnp.argmin(dist, axis=1).astype(jnp.int32)
            onehot_t = (lax.broadcasted_iota(jnp.int32, (NUM_TOKENS, CHUNK), 0)
                        == idx[None, :]).astype(jnp.bfloat16)
            hi = lax.dot_general(cbhit, onehot_t, (((1,), (0,)), ((), ())),
                                 preferred_element_type=jnp.float32)
            lo = lax.dot_general(cblot, onehot_t, (((1,), (0,)), ((), ())),
                                 preferred_element_type=jnp.float32)
            codes_ref[bb, :, cols] = hi
            codes_ref[bb, :, cols] += lo
            idx_ref[bb, 0, cols] = idx


@jax.jit
def kernel(x, codebook):
    b, c, h, w = x.shape
    hw = h * w
    xr = x.reshape(b, c, hw)
    csq = jnp.sum(codebook * codebook, axis=-1)[None, :]
    cbm2 = -2.0 * codebook
    cb_u = lax.bitcast_convert_type(codebook, jnp.uint32)
    cb_hi = lax.bitcast_convert_type(
        (cb_u >> 16).astype(jnp.uint16), jnp.bfloat16)
    cb_lo = (codebook - cb_hi.astype(jnp.float32)).astype(jnp.bfloat16)
    codes_r, idx_r = pl.pallas_call(
        _vq_block,
        grid=(b // BPS,),
        in_specs=[
            pl.BlockSpec((BPS, LATENT, hw), lambda i: (i, 0, 0)),
            pl.BlockSpec((NUM_TOKENS, LATENT), lambda i: (0, 0)),
            pl.BlockSpec((1, NUM_TOKENS), lambda i: (0, 0)),
            pl.BlockSpec((LATENT, NUM_TOKENS), lambda i: (0, 0)),
            pl.BlockSpec((LATENT, NUM_TOKENS), lambda i: (0, 0)),
        ],
        out_specs=[
            pl.BlockSpec((BPS, LATENT, hw), lambda i: (i, 0, 0)),
            pl.BlockSpec((BPS, 1, hw), lambda i: (i, 0, 0)),
        ],
        out_shape=[
            jax.ShapeDtypeStruct((b, LATENT, hw), jnp.float32),
            jax.ShapeDtypeStruct((b, 1, hw), jnp.int32),
        ],
    )(xr, cbm2, csq, cb_hi.T, cb_lo.T)
    codes = codes_r.reshape(b, c, h, w)
    indices = idx_r.reshape(b, h, w)
    return (codes, indices)


# speedup vs baseline: 1.5531x; 1.0144x over previous
import jax
import jax.numpy as jnp
from jax import lax
from jax.experimental import pallas as pl

NUM_TOKENS = 1024
LATENT = 64
SPOS = 1024
CHUNK = 256
BPS = 8


def _vq_block(xb_ref, cbm2_ref, csq_ref, cbhit_ref, cblot_ref, codes_ref, idx_ref):
    cbm2 = cbm2_ref[...]
    csq = csq_ref[...]
    cbhit = cbhit_ref[...]
    cblot = cblot_ref[...]
    for bb in range(BPS):
        for j in range(SPOS // CHUNK):
            cols = pl.ds(j * CHUNK, CHUNK)
            xb = xb_ref[bb, :, cols]
            prodm2 = lax.dot_general(xb, cbm2, (((0,), (1,)), ((), ())),
                                     preferred_element_type=jnp.float32)
            xsq = jnp.sum(xb * xb, axis=0)[:, None]
            dist = (xsq + prodm2) + csq
            idx = jnp.argmin(dist, axis=1).astype(jnp.int32)
            onehot_t = (lax.broadcasted_iota(jnp.int32, (NUM_TOKENS, CHUNK), 0)
                        == idx[None, :]).astype(jnp.bfloat16)
            hi = lax.dot_general(cbhit, onehot_t, (((1,), (0,)), ((), ())),
                                 preferred_element_type=jnp.float32)
            lo = lax.dot_general(cblot, onehot_t, (((1,), (0,)), ((), ())),
                                 preferred_element_type=jnp.float32)
            codes_ref[bb, :, cols] = hi
            codes_ref[bb, :, cols] += lo
            idx_ref[bb, 0, cols] = idx


@jax.jit
def kernel(x, codebook):
    b, c, h, w = x.shape
    hw = h * w
    xr = x.reshape(b, c, hw)
    csq = jnp.sum(codebook * codebook, axis=-1)[None, :]
    cbm2 = -2.0 * codebook
    cb_u = lax.bitcast_convert_type(codebook, jnp.uint32)
    cb_hi = lax.bitcast_convert_type(
        (cb_u >> 16).astype(jnp.uint16), jnp.bfloat16)
    cb_lo = (codebook - cb_hi.astype(jnp.float32)).astype(jnp.bfloat16)
    codes_r, idx_r = pl.pallas_call(
        _vq_block,
        grid=(b // BPS,),
        in_specs=[
            pl.BlockSpec((BPS, LATENT, hw), lambda i: (i, 0, 0)),
            pl.BlockSpec((NUM_TOKENS, LATENT), lambda i: (0, 0)),
            pl.BlockSpec((1, NUM_TOKENS), lambda i: (0, 0)),
            pl.BlockSpec((LATENT, NUM_TOKENS), lambda i: (0, 0)),
            pl.BlockSpec((LATENT, NUM_TOKENS), lambda i: (0, 0)),
        ],
        out_specs=[
            pl.BlockSpec((BPS, LATENT, hw), lambda i: (i, 0, 0)),
            pl.BlockSpec((BPS, 1, hw), lambda i: (i, 0, 0)),
        ],
        out_shape=[
            jax.ShapeDtypeStruct((b, LATENT, hw), jnp.float32),
            jax.ShapeDtypeStruct((b, 1, hw), jnp.int32),
        ],
    )(xr, cbm2, csq, cb_hi.T, cb_lo.T)
    codes = codes_r.reshape(b, c, h, w)
    indices = idx_r.reshape(b, h, w)
    return (codes, indices)
